# Initial kernel scaffold; baseline (speedup 1.0000x reference)
import jax
import jax.numpy as jnp
from jax.experimental import pallas as pl


def kernel(x, table):
    raise NotImplementedError("write your pallas kernel here")


# speedup vs baseline: 1.4597x; 1.4597x over previous
import functools

import jax
import jax.numpy as jnp
from jax import lax
from jax.experimental import pallas as pl
from jax.experimental.pallas import tpu as pltpu
from jax.experimental.pallas import tpu_sc as plsc


def _make_gather(B: int, D: int, NC: int, NS: int, chunk: int):
    NW = NC * NS
    b_per_w = B // NW
    n_chunks = b_per_w // chunk

    mesh = plsc.VectorSubcoreMesh(core_axis_name="c", subcore_axis_name="s")

    @functools.partial(
        pl.kernel,
        mesh=mesh,
        compiler_params=pltpu.CompilerParams(use_tc_tiling_on_sc=False),
        out_type=jax.ShapeDtypeStruct((B, D), jnp.float32),
        scratch_types=[
            pltpu.VMEM((chunk,), jnp.int32),
            pltpu.VMEM((chunk, D), jnp.float32),
            pltpu.SemaphoreType.DMA,
        ],
    )
    def k(idx_hbm, table_hbm, out_hbm, idx_v, rows_v, sem):
        wid = lax.axis_index("s") * NC + lax.axis_index("c")
        base = wid * b_per_w

        def body(c, carry):
            off = base + c * chunk
            pltpu.sync_copy(idx_hbm.at[pl.ds(off, chunk)], idx_v)
            pltpu.async_copy(table_hbm.at[idx_v], rows_v, sem).wait()
            pltpu.sync_copy(rows_v, out_hbm.at[pl.ds(off, chunk)])
            return carry

        lax.fori_loop(0, n_chunks, body, 0)

    return k


def kernel(x, table):
    D = table.shape[1]
    B = x.size
    idx = x.reshape(-1).astype(jnp.int32)
    info = plsc.get_sparse_core_info()
    k = _make_gather(B, D, info.num_cores, info.num_subcores, chunk=1024)
    out = k(idx, table)
    return out.reshape(x.shape + (D,))


# speedup vs baseline: 1.5041x; 1.0304x over previous
import functools

import jax
import jax.numpy as jnp
from jax import lax
from jax.experimental import pallas as pl
from jax.experimental.pallas import tpu as pltpu
from jax.experimental.pallas import tpu_sc as plsc


def _make_gather(B: int, D: int, NC: int, NS: int, chunk: int):
    NW = NC * NS
    b_per_w = B // NW
    n = b_per_w // chunk

    mesh = plsc.VectorSubcoreMesh(core_axis_name="c", subcore_axis_name="s")

    @functools.partial(
        pl.kernel,
        mesh=mesh,
        compiler_params=pltpu.CompilerParams(use_tc_tiling_on_sc=False),
        out_type=jax.ShapeDtypeStruct((B, D), jnp.float32),
        scratch_types=[
            pltpu.VMEM((2, chunk), jnp.int32),
            pltpu.VMEM((2, chunk, D), jnp.float32),
            pltpu.SemaphoreType.DMA((2,)),
            pltpu.SemaphoreType.DMA((2,)),
            pltpu.SemaphoreType.DMA((2,)),
        ],
    )
    def k(idx_hbm, table_hbm, out_hbm, idx_v, rows_v, s_idx, s_g, s_st):
        wid = lax.axis_index("s") * NC + lax.axis_index("c")
        base = wid * b_per_w

        def idx_load(c):
            b = c % 2
            return pltpu.async_copy(
                idx_hbm.at[pl.ds(base + c * chunk, chunk)], idx_v.at[b],
                s_idx.at[b])

        def gather(c):
            b = c % 2
            return pltpu.async_copy(
                table_hbm.at[idx_v.at[b]], rows_v.at[b], s_g.at[b])

        def store(c):
            b = c % 2
            return pltpu.async_copy(
                rows_v.at[b], out_hbm.at[pl.ds(base + c * chunk, chunk)],
                s_st.at[b])

        idx_cp = [idx_load(0), idx_load(1)]
        idx_cp[0].wait()
        g_cp = [gather(0), None]
        st_cp = [None, None]
        for c in range(n):
            b, b1 = c % 2, (c + 1) % 2
            if c + 1 < n:
                idx_cp[b1].wait()
                if st_cp[b1] is not None:
                    st_cp[b1].wait()
                g_cp[b1] = gather(c + 1)
            g_cp[b].wait()
            if c + 2 < n:
                idx_cp[b] = idx_load(c + 2)
            st_cp[b] = store(c)
        st_cp[(n - 2) % 2].wait()
        st_cp[(n - 1) % 2].wait()

    return k


def kernel(x, table):
    D = table.shape[1]
    B = x.size
    idx = x.reshape(-1).astype(jnp.int32)
    info = plsc.get_sparse_core_info()
    k = _make_gather(B, D, info.num_cores, info.num_subcores, chunk=1280)
    out = k(idx, table)
    return out.reshape(x.shape + (D,))
